# Initial kernel scaffold; baseline (speedup 1.0000x reference)
import jax
import jax.numpy as jnp
from jax.experimental import pallas as pl


def kernel(query, value, reference_points, spatial_shapes, level_start_index, W_so, b_so, W_aw, b_aw, W_vp, b_vp, W_op, b_op):
    raise NotImplementedError("write your pallas kernel here")


# speedup vs baseline: 60.8101x; 60.8101x over previous
import functools

import numpy as np
import jax
import jax.numpy as jnp
from jax import lax
from jax.experimental import pallas as pl
from jax.experimental.pallas import tpu as pltpu
from jax.experimental.pallas import tpu_sc as plsc

EMBED = 256
HEADS = 8
LEVELS = 4
POINTS = 4
HD = EMBED // HEADS
LP = LEVELS * POINTS
SPATIAL = [(64, 64), (32, 32), (16, 16), (8, 8)]
NV = sum(h * w for h, w in SPATIAL)
BS = 2
NQ = NV
TOT_ROWS = BS * HEADS * NV

BQ = 680
NQB = NQ // BQ

NW = 32
ITEMS = BS * NQ * HEADS
IW = ITEMS // NW
CI = 16
CH = IW // CI
ROWS_PER_CHUNK = CI * 4 * LP
IDX_ROWS = ROWS_PER_CHUNK // 128

_lvl = np.tile(np.repeat(np.arange(LEVELS), POINTS), HEADS)
_hh = np.repeat(np.arange(HEADS), LP)
_Wnp = np.array([w for (h, w) in SPATIAL], np.float32)[_lvl]
_Hnp = np.array([h for (h, w) in SPATIAL], np.float32)[_lvl]
_off = np.cumsum([0] + [h * w for h, w in SPATIAL])[:LEVELS]
_base = (_hh * NV + _off[_lvl]).astype(np.int32)
_BD = np.kron(np.eye(HEADS, dtype=np.float32),
              np.ones((LP, LP), np.float32))


def _vproj_body(x_ref, w_ref, b_ref, out_ref):
    out_ref[0, 0] = (
        jnp.dot(x_ref[0], w_ref[0], preferred_element_type=jnp.float32)
        + b_ref[0, 0])


def _prep_body(q_ref, wall_ref, ball_ref, refx_ref, refy_ref,
               wl_ref, hl_ref, base_ref, bd_ref,
               i00, i01, i10, i11, w00, w01, w10, w11):
    b = pl.program_id(0)
    q = q_ref[0]
    t = jnp.dot(q, wall_ref[...], preferred_element_type=jnp.float32) + ball_ref[0]
    sox = t[:, 0:128]
    soy = t[:, 128:256]
    awr = t[:, 256:384]
    e = jnp.exp(awr)
    s = jnp.dot(e, bd_ref[...], preferred_element_type=jnp.float32)
    aw = e / s

    wv = wl_ref[0]
    hv = hl_ref[0]
    px = refx_ref[0] * wv + sox - 0.5
    py = refy_ref[0] * hv + soy - 0.5
    x0 = jnp.floor(px)
    y0 = jnp.floor(py)
    wx1 = px - x0
    wx0 = 1.0 - wx1
    wy1 = py - y0
    wy0 = 1.0 - wy1
    vx0 = ((x0 >= 0.0) & (x0 < wv)).astype(jnp.float32)
    vx1 = ((x0 >= -1.0) & (x0 < wv - 1.0)).astype(jnp.float32)
    vy0 = ((y0 >= 0.0) & (y0 < hv)).astype(jnp.float32)
    vy1 = ((y0 >= -1.0) & (y0 < hv - 1.0)).astype(jnp.float32)
    wi = wv.astype(jnp.int32)
    xi0 = jnp.clip(x0, 0.0, wv - 1.0).astype(jnp.int32)
    xi1 = jnp.clip(x0 + 1.0, 0.0, wv - 1.0).astype(jnp.int32)
    yi0 = jnp.clip(y0, 0.0, hv - 1.0).astype(jnp.int32)
    yi1 = jnp.clip(y0 + 1.0, 0.0, hv - 1.0).astype(jnp.int32)
    bbase = b * (HEADS * NV)
    r0 = base_ref[0] + yi0 * wi + bbase
    r1 = base_ref[0] + yi1 * wi + bbase
    i00[0] = jnp.clip(r0 + xi0, 0, TOT_ROWS - 1)
    i01[0] = jnp.clip(r0 + xi1, 0, TOT_ROWS - 1)
    i10[0] = jnp.clip(r1 + xi0, 0, TOT_ROWS - 1)
    i11[0] = jnp.clip(r1 + xi1, 0, TOT_ROWS - 1)
    w00[0] = aw * (wx0 * wy0) * (vx0 * vy0)
    w01[0] = aw * (wx1 * wy0) * (vx1 * vy0)
    w10[0] = aw * (wx0 * wy1) * (vx0 * vy1)
    w11[0] = aw * (wx1 * wy1) * (vx1 * vy1)


def _post_body(s_ref, w_ref, b_ref, q_ref, out_ref):
    out_ref[0] = (
        jnp.dot(s_ref[0], w_ref[...], preferred_element_type=jnp.float32)
        + b_ref[0] + q_ref[0])


def _sc_body(table, idxh, wh, out, idx_v, w_v, rows_v, out_v, sem):
    wid = lax.axis_index("s") * 2 + lax.axis_index("c")

    def chunk(j, carry):
        pltpu.sync_copy(idxh.at[wid, j], idx_v)
        pltpu.sync_copy(wh.at[wid, j], w_v)
        cps = [
            pltpu.async_copy(table.at[idx_v.at[d]],
                             rows_v.at[pl.ds(d * 128, 128)], sem)
            for d in range(IDX_ROWS)
        ]
        for c in cps:
            c.wait()

        def item(t, c2):
            base = t * 64
            acc0 = jnp.zeros((16,), jnp.float32)
            acc1 = jnp.zeros((16,), jnp.float32)
            for g in range(4):
                wv = w_v[pl.ds(base + g * 16, 16)]
                for k in range(16):
                    r = base + g * 16 + k
                    wk = wv[k]
                    acc0 = acc0 + wk * rows_v[r, pl.ds(0, 16)]
                    acc1 = acc1 + wk * rows_v[r, pl.ds(16, 16)]
            out_v[t, pl.ds(0, 16)] = acc0
            out_v[t, pl.ds(16, 16)] = acc1
            return c2

        lax.fori_loop(0, CI, item, 0)
        pltpu.sync_copy(out_v, out.at[wid, j])
        return carry

    lax.fori_loop(0, CH, chunk, 0)


def _build_table(value, W_vp, b_vp):
    wvp_t = W_vp.reshape(HEADS, HD, EMBED).transpose(0, 2, 1)
    bvp8 = b_vp.reshape(HEADS, 1, HD)
    vtab = pl.pallas_call(
        _vproj_body,
        grid=(BS, HEADS, NQB),
        in_specs=[
            pl.BlockSpec((1, BQ, EMBED), lambda b, h, i: (b, i, 0)),
            pl.BlockSpec((1, EMBED, HD), lambda b, h, i: (h, 0, 0)),
            pl.BlockSpec((1, 1, HD), lambda b, h, i: (h, 0, 0)),
        ],
        out_specs=pl.BlockSpec((1, 1, BQ, HD), lambda b, h, i: (b, h, i, 0)),
        out_shape=jax.ShapeDtypeStruct((BS, HEADS, NV, HD), jnp.float32),
    )(value, wvp_t, bvp8)
    return vtab.reshape(TOT_ROWS, HD)


def _build_idx_w(query, reference_points, W_so, b_so, W_aw, b_aw):
    wall_t = jnp.concatenate([W_so[0::2], W_so[1::2], W_aw], axis=0).T
    ball = jnp.concatenate([b_so[0::2], b_so[1::2], b_aw]).reshape(1, 384)
    refx = jnp.tile(jnp.repeat(reference_points[..., 0], POINTS, axis=-1),
                    (1, 1, HEADS))
    refy = jnp.tile(jnp.repeat(reference_points[..., 1], POINTS, axis=-1),
                    (1, 1, HEADS))
    wl = jnp.asarray(_Wnp).reshape(1, 128)
    hl = jnp.asarray(_Hnp).reshape(1, 128)
    basev = jnp.asarray(_base).reshape(1, 128)
    bd = jnp.asarray(_BD)

    qspec = pl.BlockSpec((1, BQ, EMBED), lambda b, i: (b, i, 0))
    cspec128 = pl.BlockSpec((1, 128), lambda b, i: (0, 0))
    lspec = pl.BlockSpec((1, BQ, 128), lambda b, i: (b, i, 0))
    outs8 = pl.pallas_call(
        _prep_body,
        grid=(BS, NQB),
        in_specs=[
            qspec,
            pl.BlockSpec((EMBED, 384), lambda b, i: (0, 0)),
            pl.BlockSpec((1, 384), lambda b, i: (0, 0)),
            lspec, lspec,
            cspec128, cspec128, cspec128,
            pl.BlockSpec((128, 128), lambda b, i: (0, 0)),
        ],
        out_specs=[lspec] * 4 + [lspec] * 4,
        out_shape=(
            [jax.ShapeDtypeStruct((BS, NQ, 128), jnp.int32)] * 4
            + [jax.ShapeDtypeStruct((BS, NQ, 128), jnp.float32)] * 4
        ),
    )(query, wall_t, ball, refx, refy, wl, hl, basev, bd)
    i4 = jnp.stack(outs8[:4], axis=2)
    w4 = jnp.stack(outs8[4:], axis=2)
    idx_sc = (i4.reshape(BS, NQ, 4, HEADS, LP).transpose(0, 1, 3, 2, 4)
              .reshape(NW, CH, IDX_ROWS, 128))
    w_sc = (w4.reshape(BS, NQ, 4, HEADS, LP).transpose(0, 1, 3, 2, 4)
            .reshape(NW, CH, ROWS_PER_CHUNK))
    return idx_sc, w_sc


def _sc_gather(table, idx_sc, w_sc):
    mesh = plsc.VectorSubcoreMesh(core_axis_name="c", subcore_axis_name="s")
    f = functools.partial(
        pl.kernel,
        mesh=mesh,
        compiler_params=pltpu.CompilerParams(use_tc_tiling_on_sc=False),
        out_type=jax.ShapeDtypeStruct((NW, CH, CI, HD), jnp.float32),
        scratch_types=[
            pltpu.VMEM((IDX_ROWS, 128), jnp.int32),
            pltpu.VMEM((ROWS_PER_CHUNK,), jnp.float32),
            pltpu.VMEM((ROWS_PER_CHUNK, HD), jnp.float32),
            pltpu.VMEM((CI, HD), jnp.float32),
            pltpu.SemaphoreType.DMA,
        ],
    )(_sc_body)
    return f(table, idx_sc, w_sc)


def _post(sc_out, W_op, b_op, query):
    return pl.pallas_call(
        _post_body,
        grid=(BS, NQB),
        in_specs=[
            pl.BlockSpec((1, BQ, EMBED), lambda b, i: (b, i, 0)),
            pl.BlockSpec((EMBED, EMBED), lambda b, i: (0, 0)),
            pl.BlockSpec((1, EMBED), lambda b, i: (0, 0)),
            pl.BlockSpec((1, BQ, EMBED), lambda b, i: (b, i, 0)),
        ],
        out_specs=pl.BlockSpec((1, BQ, EMBED), lambda b, i: (b, i, 0)),
        out_shape=jax.ShapeDtypeStruct((BS, NQ, EMBED), jnp.float32),
    )(sc_out, W_op.T, b_op.reshape(1, EMBED), query)


def kernel(query, value, reference_points, spatial_shapes, level_start_index,
           W_so, b_so, W_aw, b_aw, W_vp, b_vp, W_op, b_op):
    table = _build_table(value, W_vp, b_vp)
    idx_sc, w_sc = _build_idx_w(query, reference_points, W_so, b_so, W_aw, b_aw)
    sc_out = _sc_gather(table, idx_sc, w_sc)
    sc_out = sc_out.reshape(BS, NQ, EMBED)
    return _post(sc_out, W_op, b_op, query)


# speedup vs baseline: 129.3143x; 2.1265x over previous
import functools

import numpy as np
import jax
import jax.numpy as jnp
from jax import lax
from jax.experimental import pallas as pl
from jax.experimental.pallas import tpu as pltpu
from jax.experimental.pallas import tpu_sc as plsc

EMBED = 256
HEADS = 8
LEVELS = 4
POINTS = 4
HD = EMBED // HEADS
LP = LEVELS * POINTS
SPATIAL = [(64, 64), (32, 32), (16, 16), (8, 8)]
NV = sum(h * w for h, w in SPATIAL)
BS = 2
NQ = NV
TOT_ROWS = BS * NV * HEADS

BQ = 680
NQB = NQ // BQ

NW = 32
ITEMS = BS * NQ * HEADS
IW = ITEMS // NW
CI = 16
CH = IW // CI
ROWS_PER_CHUNK = CI * 4 * LP
IDX_ROWS = ROWS_PER_CHUNK // 128

_lvl = np.tile(np.repeat(np.arange(LEVELS), POINTS), HEADS)
_hh = np.repeat(np.arange(HEADS), LP)
_Wnp = np.array([w for (h, w) in SPATIAL], np.float32)[_lvl]
_Hnp = np.array([h for (h, w) in SPATIAL], np.float32)[_lvl]
_off = np.cumsum([0] + [h * w for h, w in SPATIAL])[:LEVELS]
_base = _off[_lvl].astype(np.int32)
_hlane = _hh.astype(np.int32)
_BD = np.kron(np.eye(HEADS, dtype=np.float32),
              np.ones((LP, LP), np.float32))


def _vproj_body(x_ref, w_ref, b_ref, out_ref):
    out_ref[0] = (
        jnp.dot(x_ref[0], w_ref[...], preferred_element_type=jnp.float32)
        + b_ref[0])


def _prep_body(q_ref, wall_ref, ball_ref, refx_ref, refy_ref,
               wl_ref, hl_ref, base_ref, hlane_ref, bd_ref,
               idx_out, w_out):
    b = pl.program_id(0)
    q = q_ref[0]
    t = jnp.dot(q, wall_ref[...], preferred_element_type=jnp.float32) + ball_ref[0]
    sox = t[:, 0:128]
    soy = t[:, 128:256]
    awr = t[:, 256:384]
    e = jnp.exp(awr)
    s = jnp.dot(e, bd_ref[...], preferred_element_type=jnp.float32)
    aw = e / s

    wv = wl_ref[0]
    hv = hl_ref[0]
    px = refx_ref[0] * wv + sox - 0.5
    py = refy_ref[0] * hv + soy - 0.5
    x0 = jnp.floor(px)
    y0 = jnp.floor(py)
    wx1 = px - x0
    wx0 = 1.0 - wx1
    wy1 = py - y0
    wy0 = 1.0 - wy1
    vx0 = ((x0 >= 0.0) & (x0 < wv)).astype(jnp.float32)
    vx1 = ((x0 >= -1.0) & (x0 < wv - 1.0)).astype(jnp.float32)
    vy0 = ((y0 >= 0.0) & (y0 < hv)).astype(jnp.float32)
    vy1 = ((y0 >= -1.0) & (y0 < hv - 1.0)).astype(jnp.float32)
    wi = wv.astype(jnp.int32)
    xi0 = jnp.clip(x0, 0.0, wv - 1.0).astype(jnp.int32)
    xi1 = jnp.clip(x0 + 1.0, 0.0, wv - 1.0).astype(jnp.int32)
    yi0 = jnp.clip(y0, 0.0, hv - 1.0).astype(jnp.int32)
    yi1 = jnp.clip(y0 + 1.0, 0.0, hv - 1.0).astype(jnp.int32)
    bbase = b * NV
    r0 = base_ref[0] + yi0 * wi + bbase
    r1 = base_ref[0] + yi1 * wi + bbase
    hl8 = hlane_ref[0]
    idx_out[0, :, 0] = jnp.clip((r0 + xi0) * HEADS + hl8, 0, TOT_ROWS - 1)
    idx_out[0, :, 1] = jnp.clip((r0 + xi1) * HEADS + hl8, 0, TOT_ROWS - 1)
    idx_out[0, :, 2] = jnp.clip((r1 + xi0) * HEADS + hl8, 0, TOT_ROWS - 1)
    idx_out[0, :, 3] = jnp.clip((r1 + xi1) * HEADS + hl8, 0, TOT_ROWS - 1)
    w_out[0, :, 0] = aw * (wx0 * wy0) * (vx0 * vy0)
    w_out[0, :, 1] = aw * (wx1 * wy0) * (vx1 * vy0)
    w_out[0, :, 2] = aw * (wx0 * wy1) * (vx0 * vy1)
    w_out[0, :, 3] = aw * (wx1 * wy1) * (vx1 * vy1)


def _post_body(s_ref, w_ref, b_ref, q_ref, out_ref):
    out_ref[0] = (
        jnp.dot(s_ref[0], w_ref[...], preferred_element_type=jnp.float32)
        + b_ref[0] + q_ref[0])


def _sc_body(table, idxh, wh, out, idx_v, w_v, rows_v, out_v, semg0, semg1, sem_o):
    wid = lax.axis_index("s") * 2 + lax.axis_index("c")
    semg = [semg0, semg1]

    def load_and_fire(j, par):
        pltpu.sync_copy(idxh.at[wid, j], idx_v.at[par])
        pltpu.sync_copy(wh.at[wid, j], w_v.at[par])
        for d in range(IDX_ROWS):
            pltpu.async_copy(table.at[idx_v.at[par, d]],
                             rows_v.at[par, pl.ds(d * 128, 128)], semg[par])

    def drain(par):
        for d in range(IDX_ROWS):
            pltpu.make_async_copy(
                table.at[idx_v.at[par, d]],
                rows_v.at[par, pl.ds(d * 128, 128)], semg[par]).wait()

    load_and_fire(0, 0)

    def outer(j2, carry):
        for par in range(2):
            j = j2 * 2 + par

            @pl.when(j < CH - 1)
            def _():
                load_and_fire(j + 1, 1 - par)

            drain(par)

            def item(t, c2):
                qq = t >> 3
                h = t & 7
                acc0 = jnp.zeros((16,), jnp.float32)
                acc1 = jnp.zeros((16,), jnp.float32)
                for g in range(4):
                    row8 = qq * 4 + g
                    col = h * 16
                    wv16 = w_v[par, row8, pl.ds(col, 16)]
                    rbase = row8 * 128 + col
                    for k in range(16):
                        wk = wv16[k]
                        acc0 = acc0 + wk * rows_v[par, rbase + k, pl.ds(0, 16)]
                        acc1 = acc1 + wk * rows_v[par, rbase + k, pl.ds(16, 16)]
                out_v[t, pl.ds(0, 16)] = acc0
                out_v[t, pl.ds(16, 16)] = acc1
                return c2

            lax.fori_loop(0, CI, item, 0)
            cp = pltpu.async_copy(out_v, out.at[wid, j], sem_o)
            cp.wait()
        return carry

    lax.fori_loop(0, CH // 2, outer, 0)


def _build_table(value, W_vp, b_vp):
    vproj = pl.pallas_call(
        _vproj_body,
        grid=(BS, NQB),
        in_specs=[
            pl.BlockSpec((1, BQ, EMBED), lambda b, i: (b, i, 0)),
            pl.BlockSpec((EMBED, EMBED), lambda b, i: (0, 0)),
            pl.BlockSpec((1, EMBED), lambda b, i: (0, 0)),
        ],
        out_specs=pl.BlockSpec((1, BQ, EMBED), lambda b, i: (b, i, 0)),
        out_shape=jax.ShapeDtypeStruct((BS, NV, EMBED), jnp.float32),
    )(value, W_vp.T, b_vp.reshape(1, EMBED))
    return vproj.reshape(TOT_ROWS, HD)


def _build_idx_w(query, reference_points, W_so, b_so, W_aw, b_aw):
    wall_t = jnp.concatenate([W_so[0::2], W_so[1::2], W_aw], axis=0).T
    ball = jnp.concatenate([b_so[0::2], b_so[1::2], b_aw]).reshape(1, 384)
    refx = jnp.tile(jnp.repeat(reference_points[..., 0], POINTS, axis=-1),
                    (1, 1, HEADS))
    refy = jnp.tile(jnp.repeat(reference_points[..., 1], POINTS, axis=-1),
                    (1, 1, HEADS))
    wl = jnp.asarray(_Wnp).reshape(1, 128)
    hl = jnp.asarray(_Hnp).reshape(1, 128)
    basev = jnp.asarray(_base).reshape(1, 128)
    hlane = jnp.asarray(_hlane).reshape(1, 128)
    bd = jnp.asarray(_BD)

    qspec = pl.BlockSpec((1, BQ, EMBED), lambda b, i: (b, i, 0))
    cspec128 = pl.BlockSpec((1, 128), lambda b, i: (0, 0))
    lspec = pl.BlockSpec((1, BQ, 128), lambda b, i: (b, i, 0))
    ospec = pl.BlockSpec((1, BQ, 4, 128), lambda b, i: (b, i, 0, 0))
    idx4, w4 = pl.pallas_call(
        _prep_body,
        grid=(BS, NQB),
        in_specs=[
            qspec,
            pl.BlockSpec((EMBED, 384), lambda b, i: (0, 0)),
            pl.BlockSpec((1, 384), lambda b, i: (0, 0)),
            lspec, lspec,
            cspec128, cspec128, cspec128, cspec128,
            pl.BlockSpec((128, 128), lambda b, i: (0, 0)),
        ],
        out_specs=[ospec, ospec],
        out_shape=[
            jax.ShapeDtypeStruct((BS, NQ, 4, 128), jnp.int32),
            jax.ShapeDtypeStruct((BS, NQ, 4, 128), jnp.float32),
        ],
    )(query, wall_t, ball, refx, refy, wl, hl, basev, hlane, bd)
    idx_sc = idx4.reshape(NW, CH, IDX_ROWS, 128)
    w_sc = w4.reshape(NW, CH, IDX_ROWS, 128)
    return idx_sc, w_sc


def _sc_gather(table, idx_sc, w_sc):
    mesh = plsc.VectorSubcoreMesh(core_axis_name="c", subcore_axis_name="s")
    f = functools.partial(
        pl.kernel,
        mesh=mesh,
        compiler_params=pltpu.CompilerParams(use_tc_tiling_on_sc=False),
        out_type=jax.ShapeDtypeStruct((NW, CH, CI, HD), jnp.float32),
        scratch_types=[
            pltpu.VMEM((2, IDX_ROWS, 128), jnp.int32),
            pltpu.VMEM((2, IDX_ROWS, 128), jnp.float32),
            pltpu.VMEM((2, ROWS_PER_CHUNK, HD), jnp.float32),
            pltpu.VMEM((CI, HD), jnp.float32),
            pltpu.SemaphoreType.DMA,
            pltpu.SemaphoreType.DMA,
            pltpu.SemaphoreType.DMA,
        ],
    )(_sc_body)
    return f(table, idx_sc, w_sc)


def _post(sc_out, W_op, b_op, query):
    return pl.pallas_call(
        _post_body,
        grid=(BS, NQB),
        in_specs=[
            pl.BlockSpec((1, BQ, EMBED), lambda b, i: (b, i, 0)),
            pl.BlockSpec((EMBED, EMBED), lambda b, i: (0, 0)),
            pl.BlockSpec((1, EMBED), lambda b, i: (0, 0)),
            pl.BlockSpec((1, BQ, EMBED), lambda b, i: (b, i, 0)),
        ],
        out_specs=pl.BlockSpec((1, BQ, EMBED), lambda b, i: (b, i, 0)),
        out_shape=jax.ShapeDtypeStruct((BS, NQ, EMBED), jnp.float32),
    )(sc_out, W_op.T, b_op.reshape(1, EMBED), query)


def kernel(query, value, reference_points, spatial_shapes, level_start_index,
           W_so, b_so, W_aw, b_aw, W_vp, b_vp, W_op, b_op):
    table = _build_table(value, W_vp, b_vp)
    idx_sc, w_sc = _build_idx_w(query, reference_points, W_so, b_so, W_aw, b_aw)
    sc_out = _sc_gather(table, idx_sc, w_sc)
    sc_out = sc_out.reshape(BS, NQ, EMBED)
    return _post(sc_out, W_op, b_op, query)
